# baseline (speedup 1.0000x reference)
import jax
import jax.numpy as jnp
from jax import lax
from jax.experimental import pallas as pl
from jax.experimental.pallas import tpu as pltpu
from jax.experimental.pallas import tpu_sc as plsc

D_MODEL = 64
SEQ = 200
BATCH = 4096
LANES = 16
NUM_CORES = 2
NUM_SUBCORES = 16
NW = NUM_CORES * NUM_SUBCORES
B_PER_W = BATCH // NW
IDXN = 208
NBUF = 2
TABLE_PAIRS = 500000
NGRP = SEQ // LANES


def _positional_encoding(length, depth):
    half = depth // 2
    positions = jnp.arange(length, dtype=jnp.float32)[:, None]
    depths = jnp.arange(half, dtype=jnp.float32)[None, :]
    angle_rates = 1.0 / (10000.0 ** depths)
    angle_rads = positions * angle_rates
    return jnp.concatenate([jnp.sin(angle_rads), jnp.cos(angle_rads)], axis=-1)


def _lane_bcast(vec, j):
    return lax.gather(
        vec, jnp.full((LANES, 1), j, jnp.int32),
        dimension_numbers=lax.GatherDimensionNumbers(
            offset_dims=(), collapsed_slice_dims=(0,), start_index_map=(0,)),
        slice_sizes=(1,),
        mode=lax.GatherScatterMode.PROMISE_IN_BOUNDS)


def _sc_body(table_hbm, x_hbm, pe_hbm, out_hbm,
             idx_v, pe_v, idx2_v, gath_v, outb_v, gsems):
    wid = lax.axis_index("s") * NUM_CORES + lax.axis_index("c")
    b0 = wid * B_PER_W
    pltpu.sync_copy(x_hbm.at[wid], idx_v.at[pl.ds(0, B_PER_W * SEQ)])
    pltpu.sync_copy(pe_hbm, pe_v)
    iota = lax.iota(jnp.int32, LANES)

    def prep_chunk(g, p):
        f0 = g * SEQ

        @plsc.parallel_loop(0, NGRP, unroll=4)
        def _grp(rg):
            r0 = rg * LANES
            idx2_v[p][pl.ds(r0, LANES)] = idx_v[pl.ds(f0 + r0, LANES)] >> 1

        piv = jnp.clip(idx_v[pl.ds(f0 + NGRP * LANES, LANES)] >> 1,
                       0, TABLE_PAIRS - 1)
        idx2_v[p][pl.ds(NGRP * LANES, LANES)] = piv

    def start_gather(p):
        pltpu.async_copy(table_hbm.at[idx2_v[p].at[pl.ds(0, 128)]],
                         gath_v[p].at[pl.ds(0, 128)], gsems[p])
        pltpu.async_copy(table_hbm.at[idx2_v[p].at[pl.ds(128, IDXN - 128)]],
                         gath_v[p].at[pl.ds(128, IDXN - 128)], gsems[p])

    def wait_gather(p):
        pltpu.make_async_copy(table_hbm.at[idx2_v[p]], gath_v[p],
                              gsems[p]).wait()

    def compute(g, p):
        gath, outb = gath_v[p], outb_v
        f0 = g * SEQ

        def do_rows(r0, nj):
            offv = (idx_v[pl.ds(f0 + r0, LANES)] & 1) << 6
            for j in range(nj):
                r = r0 + j
                offs = _lane_bcast(offv, j)
                rows = jnp.full((LANES,), r, jnp.int32)
                for c in range(0, D_MODEL, LANES):
                    val = plsc.load_gather(gath, [rows, offs + (c + iota)])
                    outb[r, pl.ds(c, LANES)] = (
                        val * 8.0 + pe_v[r, pl.ds(c, LANES)])

        @plsc.parallel_loop(0, NGRP, unroll=4)
        def _grp(rg):
            do_rows(rg * LANES, LANES)

        do_rows(NGRP * LANES, SEQ - NGRP * LANES)

    prep_chunk(0, 0)
    start_gather(0)

    @pl.loop(0, B_PER_W // NBUF)
    def _ring(h):
        for p in range(NBUF):
            g = h * NBUF + p
            wait_gather(p)
            nxt = (p + 1) % NBUF

            @pl.when(g + 1 < B_PER_W)
            def _():
                prep_chunk(g + 1, nxt)
                start_gather(nxt)

            compute(g, p)
            pltpu.sync_copy(outb_v, out_hbm.at[b0 + g])


def kernel(x, table):
    table2 = table.reshape(table.shape[0] // 2, 128)
    x2 = x.reshape(NW, B_PER_W * SEQ)
    pe = _positional_encoding(SEQ, D_MODEL)

    mesh = plsc.VectorSubcoreMesh(
        core_axis_name="c", subcore_axis_name="s",
        num_cores=NUM_CORES, num_subcores=NUM_SUBCORES,
    )
    k = pl.kernel(
        _sc_body,
        out_type=jax.ShapeDtypeStruct((BATCH, SEQ, D_MODEL), jnp.float32),
        mesh=mesh,
        scratch_types=[
            pltpu.VMEM((B_PER_W * SEQ + LANES,), jnp.int32),
            pltpu.VMEM((SEQ, D_MODEL), jnp.float32),
            [pltpu.VMEM((IDXN,), jnp.int32) for _ in range(NBUF)],
            [pltpu.VMEM((IDXN, 128), jnp.float32) for _ in range(NBUF)],
            pltpu.VMEM((SEQ, D_MODEL), jnp.float32),
            [pltpu.SemaphoreType.DMA for _ in range(NBUF)],
        ],
        compiler_params=pltpu.CompilerParams(needs_layout_passes=False),
    )
    return k(table2, x2, pe)


# speedup vs baseline: 1.0562x; 1.0562x over previous
import jax
import jax.numpy as jnp
from jax import lax
from jax.experimental import pallas as pl
from jax.experimental.pallas import tpu as pltpu
from jax.experimental.pallas import tpu_sc as plsc

D_MODEL = 64
SEQ = 200
BATCH = 4096
LANES = 16
NUM_CORES = 2
NUM_SUBCORES = 16
NW = NUM_CORES * NUM_SUBCORES
B_PER_W = BATCH // NW
IDXN = 208
NBUF = 2
TABLE_PAIRS = 500000
NGRP = SEQ // LANES


def _positional_encoding(length, depth):
    half = depth // 2
    positions = jnp.arange(length, dtype=jnp.float32)[:, None]
    depths = jnp.arange(half, dtype=jnp.float32)[None, :]
    angle_rates = 1.0 / (10000.0 ** depths)
    angle_rads = positions * angle_rates
    return jnp.concatenate([jnp.sin(angle_rads), jnp.cos(angle_rads)], axis=-1)


def _lane_bcast(vec, j):
    return lax.gather(
        vec, jnp.full((LANES, 1), j, jnp.int32),
        dimension_numbers=lax.GatherDimensionNumbers(
            offset_dims=(), collapsed_slice_dims=(0,), start_index_map=(0,)),
        slice_sizes=(1,),
        mode=lax.GatherScatterMode.PROMISE_IN_BOUNDS)


def _sc_body(table_hbm, x_hbm, pe_hbm, out_hbm,
             idx_v, pe_v, idx2_v, gath_v, outb_v, gsems):
    wid = lax.axis_index("s") * NUM_CORES + lax.axis_index("c")
    b0 = wid * B_PER_W
    pltpu.sync_copy(x_hbm.at[wid], idx_v.at[pl.ds(0, B_PER_W * SEQ)])
    pltpu.sync_copy(pe_hbm, pe_v)
    iota = lax.iota(jnp.int32, LANES)

    def prep_chunk(g, p):
        f0 = g * SEQ

        @plsc.parallel_loop(0, NGRP, unroll=4)
        def _grp(rg):
            r0 = rg * LANES
            idx2_v[p][pl.ds(r0, LANES)] = idx_v[pl.ds(f0 + r0, LANES)] >> 1

        piv = jnp.clip(idx_v[pl.ds(f0 + NGRP * LANES, LANES)] >> 1,
                       0, TABLE_PAIRS - 1)
        idx2_v[p][pl.ds(NGRP * LANES, LANES)] = piv

    def start_gather(p):
        pltpu.async_copy(table_hbm.at[idx2_v[p].at[pl.ds(0, 128)]],
                         gath_v[p].at[pl.ds(0, 128)], gsems[p])
        pltpu.async_copy(table_hbm.at[idx2_v[p].at[pl.ds(128, IDXN - 128)]],
                         gath_v[p].at[pl.ds(128, IDXN - 128)], gsems[p])

    def wait_gather(p):
        pltpu.make_async_copy(table_hbm.at[idx2_v[p]], gath_v[p],
                              gsems[p]).wait()

    def compute(g, p):
        gath, outb = gath_v[p], outb_v
        f0 = g * SEQ

        def do_rows(r0, nj):
            offv = (idx_v[pl.ds(f0 + r0, LANES)] & 1) << 6
            for j in range(nj):
                r = r0 + j
                offs = _lane_bcast(offv, j)
                rows = jnp.full((LANES,), r, jnp.int32)
                for c in range(0, D_MODEL, LANES):
                    val = plsc.load_gather(gath, [rows, offs + (c + iota)])
                    outb[r, pl.ds(c, LANES)] = (
                        val * 8.0 + pe_v[r, pl.ds(c, LANES)])

        @plsc.parallel_loop(0, NGRP, unroll=2)
        def _grp(rg):
            do_rows(rg * LANES, LANES)

        do_rows(NGRP * LANES, SEQ - NGRP * LANES)

    prep_chunk(0, 0)
    start_gather(0)

    @pl.loop(0, B_PER_W // NBUF)
    def _ring(h):
        for p in range(NBUF):
            g = h * NBUF + p
            wait_gather(p)
            nxt = (p + 1) % NBUF

            @pl.when(g + 1 < B_PER_W)
            def _():
                prep_chunk(g + 1, nxt)
                start_gather(nxt)

            compute(g, p)
            pltpu.sync_copy(outb_v, out_hbm.at[b0 + g])


def kernel(x, table):
    table2 = table.reshape(table.shape[0] // 2, 128)
    x2 = x.reshape(NW, B_PER_W * SEQ)
    pe = _positional_encoding(SEQ, D_MODEL)

    mesh = plsc.VectorSubcoreMesh(
        core_axis_name="c", subcore_axis_name="s",
        num_cores=NUM_CORES, num_subcores=NUM_SUBCORES,
    )
    k = pl.kernel(
        _sc_body,
        out_type=jax.ShapeDtypeStruct((BATCH, SEQ, D_MODEL), jnp.float32),
        mesh=mesh,
        scratch_types=[
            pltpu.VMEM((B_PER_W * SEQ + LANES,), jnp.int32),
            pltpu.VMEM((SEQ, D_MODEL), jnp.float32),
            [pltpu.VMEM((IDXN,), jnp.int32) for _ in range(NBUF)],
            [pltpu.VMEM((IDXN, 128), jnp.float32) for _ in range(NBUF)],
            pltpu.VMEM((SEQ, D_MODEL), jnp.float32),
            [pltpu.SemaphoreType.DMA for _ in range(NBUF)],
        ],
        compiler_params=pltpu.CompilerParams(needs_layout_passes=False),
    )
    return k(table2, x2, pe)
